# baseline (speedup 1.0000x reference)
import jax
import jax.numpy as jnp
from jax import lax
from jax.experimental import pallas as pl
from jax.experimental.pallas import tpu as pltpu
from jax.experimental.pallas import tpu_sc as plsc

B = 4
N_NODES = 10000
E = 160000
L = 50
D = 128
H = 128
HEADS = 3
R = 5
NB = 4
VOCAB = 100000

NC = 2
NS = 16
NW = NC * NS
EPT = E // NW
FULLG = EPT // 16
TAIL = EPT - FULLG * 16
WIN = 5120

SROWS = 304
XBASE = 1 + R * L
CAPL = 32
LP = 64


def _sc_body(ei_hbm, et_hbm, nids_hbm, sent_hbm, emb_hbm, s_out, deg_out,
             marker, nids0, nids1, srcb0, srcb1, dstb0, dstb1, etb0, etb1,
             sentb0, sentb1, idx_s, idx_o, idx_m,
             rowbuf, ones_r, zrows, z16, sg, degg, sem_st, sem_a, sem_b):
    nids = (nids0, nids1)
    srcb = (srcb0, srcb1)
    dstb = (dstb0, dstb1)
    etb = (etb0, etb1)
    sentb = (sentb0, sentb1)
    cc = lax.axis_index("c")
    ss = lax.axis_index("s")
    wid = cc * NS + ss

    zero16f = jnp.zeros((16,), jnp.float32)
    one16f = jnp.ones((16,), jnp.float32)
    zero16i = jnp.zeros((16,), jnp.int32)
    neg16i = jnp.full((16,), -1, jnp.int32)
    lane = lax.iota(jnp.int32, 16)

    abase = pl.multiple_of((wid * EPT >> 7) << 7, 128)
    eoff = wid * EPT - abase

    def stage(g, slot):
        descs = [
            pltpu.async_copy(ei_hbm.at[g, 0, pl.ds(abase, WIN)],
                             srcb[slot].at[pl.ds(0, WIN)], sem_st),
            pltpu.async_copy(ei_hbm.at[g, 1, pl.ds(abase, WIN)],
                             dstb[slot].at[pl.ds(0, WIN)], sem_st),
            pltpu.async_copy(et_hbm.at[g, pl.ds(abase, WIN)],
                             etb[slot].at[pl.ds(0, WIN)], sem_st),
            pltpu.async_copy(nids_hbm.at[g], nids[slot], sem_st),
            pltpu.async_copy(sent_hbm.at[g], sentb[slot], sem_st),
        ]
        return descs

    def stage_wait(descs):
        for d in descs:
            d.wait()

    descs = stage(0, 0)

    def _init_const(i, _):
        for k in range(8):
            zrows[i, pl.ds(k * 16, 16)] = zero16f
        z16[i, pl.ds(0, 16)] = zero16f
        ones_r[i, pl.ds(0, 16)] = one16f
        ones_r[i + 16, pl.ds(0, 16)] = one16f
        return 0
    lax.fori_loop(0, 16, _init_const, 0)

    def _memset(i, _):
        marker[pl.ds(i * 16, 16)] = neg16i
        return 0
    lax.fori_loop(0, N_NODES // 16, _memset, 0)

    idx = 0
    for g in range(B):
        for ch in range(SROWS // 16):
            @pl.when(ss == (idx % NS))
            def _():
                pltpu.sync_copy(zrows, sg.at[g, pl.ds(ch * 16, 16)])
            idx += 1
    for g in range(B):
        for ch in range(LP // 16):
            @pl.when(ss == ((g * (LP // 16) + ch) % NS))
            def _():
                pltpu.sync_copy(z16, degg.at[g, pl.ds(ch * 16, 16)])

    stage_wait(descs)
    plsc.subcore_barrier()

    for g in range(B):
        slot = g % 2
        tv = dstb[slot][pl.ds(eoff + FULLG * 16, 16)]
        dstb[slot][pl.ds(eoff + FULLG * 16, 16)] = jnp.where(lane < TAIL, tv, 0)
        sv = srcb[slot][pl.ds(eoff + FULLG * 16, 16)]
        srcb[slot][pl.ds(eoff + FULLG * 16, 16)] = jnp.where(lane < TAIL, sv, 0)

        for k in range(LP // 16):
            se = sentb[slot][pl.ds(k * 16, 16)]
            lv = lane + k * 16
            plsc.store_scatter(marker, [se], lv, mask=lv < L)

        def _prefill(ch, _):
            for k in range(2):
                idx_s[ch, pl.ds(k * 16, 16)] = zero16i
                idx_o[ch, pl.ds(k * 16, 16)] = zero16i
                idx_m[ch, pl.ds(k * 16, 16)] = zero16i
            return 0
        lax.fori_loop(0, CAPL, _prefill, 0)

        def _scan_group(j, cnt, col, tail_mask=None):
            d = dstb[slot][pl.ds(eoff + j * 16, 16)]
            s = srcb[slot][pl.ds(eoff + j * 16, 16)]
            t = etb[slot][pl.ds(eoff + j * 16, 16)]
            m = plsc.load_gather(marker, [d])
            valid = m >= 0
            if tail_mask is not None:
                valid = valid & tail_mask
            nid = plsc.load_gather(nids[slot], [s])
            oidx = 1 + t * L + m
            midx = 1 + m
            pos = jnp.minimum(cnt, CAPL - 1)
            plsc.store_scatter(idx_s, [pos, col], nid, mask=valid)
            plsc.store_scatter(idx_o, [pos, col], oidx, mask=valid)
            plsc.store_scatter(idx_m, [pos, col], midx, mask=valid)
            return jnp.minimum(cnt + valid.astype(jnp.int32), CAPL)

        lane16 = lane + 16

        def _scan_pair(p, carry):
            ca, cb = carry
            ca = _scan_group(2 * p, ca, lane)
            cb = _scan_group(2 * p + 1, cb, lane16)
            return (ca, cb)

        cnt_a, cnt_b = lax.fori_loop(
            0, FULLG // 2, _scan_pair,
            (jnp.zeros((16,), jnp.int32), jnp.zeros((16,), jnp.int32)))
        cnt_a = _scan_group(FULLG, cnt_a, lane, tail_mask=lane < TAIL)

        for k in range(LP // 16):
            se = sentb[slot][pl.ds(k * 16, 16)]
            lv = lane + k * 16
            plsc.store_scatter(marker, [se], neg16i, mask=lv < L)

        t0 = wid == 0
        for k in range(LP // 16):
            se = sentb[slot][pl.ds(k * 16, 16)]
            lv = lane + k * 16
            valid = (lv < L) & t0
            nid = plsc.load_gather(nids[slot], [se])
            pos = jnp.minimum(cnt_a, CAPL - 1)
            plsc.store_scatter(idx_s, [pos, lane], nid, mask=valid)
            plsc.store_scatter(idx_o, [pos, lane], XBASE + lv, mask=valid)
            plsc.store_scatter(idx_m, [pos, lane], zero16i, mask=valid)
            cnt_a = jnp.minimum(cnt_a + valid.astype(jnp.int32), CAPL)

        if g + 1 < B:
            descs = stage(g + 1, (g + 1) % 2)

        nmax = jnp.maximum(jnp.max(cnt_a), jnp.max(cnt_b))

        def _flush(j, _):
            pltpu.async_copy(emb_hbm.at[idx_s.at[j]], rowbuf, sem_a).wait()
            pltpu.sync_copy(rowbuf, sg.at[g].at[idx_o.at[j]], add=True)
            pltpu.sync_copy(ones_r, degg.at[g].at[idx_m.at[j]], add=True)
            return 0
        lax.fori_loop(0, nmax, _flush, 0)

        if g + 1 < B:
            stage_wait(descs)

    plsc.subcore_barrier()

    idx = 0
    for g in range(B):
        for ch in range(SROWS // 16):
            @pl.when(ss == (idx % NS))
            def _():
                pltpu.sync_copy(sg.at[g, pl.ds(ch * 16, 16)],
                                s_out.at[cc, g, pl.ds(ch * 16, 16)])
            idx += 1
    for g in range(B):
        for ch in range(LP // 16):
            @pl.when(ss == ((g * (LP // 16) + ch) % NS))
            def _():
                pltpu.sync_copy(degg.at[g, pl.ds(ch * 16, 16)],
                                deg_out.at[cc, g, pl.ds(ch * 16, 16)])


@jax.jit
def _sc_stage(ei, et, nids, sent, emb):
    mesh = plsc.VectorSubcoreMesh(core_axis_name="c", subcore_axis_name="s")
    f = pl.kernel(
        _sc_body,
        out_type=[
            jax.ShapeDtypeStruct((NC, B, SROWS, D), jnp.float32),
            jax.ShapeDtypeStruct((NC, B, LP, 16), jnp.float32),
        ],
        mesh=mesh,
        compiler_params=pltpu.CompilerParams(needs_layout_passes=False),
        scratch_types=[
            pltpu.VMEM((N_NODES,), jnp.int32),
            pltpu.VMEM((N_NODES,), jnp.int32),
            pltpu.VMEM((N_NODES,), jnp.int32),
            pltpu.VMEM((WIN + 16,), jnp.int32),
            pltpu.VMEM((WIN + 16,), jnp.int32),
            pltpu.VMEM((WIN + 16,), jnp.int32),
            pltpu.VMEM((WIN + 16,), jnp.int32),
            pltpu.VMEM((WIN + 16,), jnp.int32),
            pltpu.VMEM((WIN + 16,), jnp.int32),
            pltpu.VMEM((LP,), jnp.int32),
            pltpu.VMEM((LP,), jnp.int32),
            pltpu.VMEM((CAPL, 32), jnp.int32),
            pltpu.VMEM((CAPL, 32), jnp.int32),
            pltpu.VMEM((CAPL, 32), jnp.int32),
            pltpu.VMEM((32, D), jnp.float32),
            pltpu.VMEM((32, 16), jnp.float32),
            pltpu.VMEM((16, D), jnp.float32),
            pltpu.VMEM((16, 16), jnp.float32),
            pltpu.VMEM_SHARED((B, SROWS, D), jnp.float32),
            pltpu.VMEM_SHARED((B, LP, 16), jnp.float32),
            pltpu.SemaphoreType.DMA,
            pltpu.SemaphoreType.DMA,
            pltpu.SemaphoreType.DMA,
        ],
    )
    return f(ei, et, nids, sent, emb)


def _tc_body(s_ref, deg_ref, sent_ref, basis_ref, comp_ref, root_ref, bias_ref,
             wq_ref, wk_ref, wv_ref, attw_ref, attb_ref, fcw_ref, fcb_ref,
             out_ref):
    s = s_ref[0] + s_ref[1]
    deg = jnp.sum(deg_ref[0] + deg_ref[1], axis=-1) * (1.0 / 16.0)

    w = jnp.einsum("rb,bde->rde", comp_ref[...], basis_ref[...])
    msum = jnp.zeros((B, L, D), jnp.float32)
    for r in range(R):
        msum = msum + jnp.einsum("gld,de->gle", s[:, 1 + r * L:1 + (r + 1) * L, :], w[r])

    sent = sent_ref[...][:, :L]
    p = (sent[:, :, None] == sent[:, None, :]).astype(jnp.float32)
    sum_s = jnp.einsum("glk,gkd->gld", p, msum)
    sum_deg = jnp.einsum("glk,gk->gl", p, deg[:, 1:1 + L])
    agg = sum_s / jnp.clip(sum_deg, 1.0, None)[..., None]

    xs = s[:, XBASE:XBASE + L, :]
    x = jnp.einsum("gld,de->gle", xs, root_ref[...]) + agg + bias_ref[...]

    def heads(wref):
        hh = jnp.einsum("gld,hde->glhe", x, wref[...])
        return hh.reshape(B, L, HEADS * H)
    q = heads(wq_ref)
    k = heads(wk_ref)
    v = heads(wv_ref)
    score = jnp.einsum("gle,gme->glm", q, k) * (1.0 / jnp.sqrt(float(HEADS * H)))
    score = jax.nn.softmax(score, axis=-1)
    hidden = jnp.einsum("glm,gme->gle", score, v)
    attn = jnp.einsum("gle,eo->glo", hidden, attw_ref[...]) + attb_ref[...]
    pooled = jnp.sum(hidden * attn, axis=1)
    logits = pooled @ fcw_ref[...] + fcb_ref[...]
    out_ref[...] = 1.0 / (1.0 + jnp.exp(-logits))


@jax.jit
def _tc_stage(s, deg, sent, basis, comp, root, bias, wq, wk, wv, attw, attb,
              fcw, fcb):
    return pl.pallas_call(
        _tc_body,
        out_shape=jax.ShapeDtypeStruct((B, 1), jnp.float32),
    )(s, deg, sent, basis, comp, root, bias, wq, wk, wv, attw, attb, fcw, fcb)


def kernel(sent_ids, edge_index, edge_type, node_ids, emb, basis, comp, root,
           bias, WQ, WK, WV, attW, attb, fcW, fcb):
    ei = edge_index.astype(jnp.int32)
    et = edge_type.astype(jnp.int32)
    sent = jnp.pad(sent_ids.astype(jnp.int32), ((0, 0), (0, LP - L)))
    nids = node_ids.astype(jnp.int32)
    s, deg = _sc_stage(ei, et, nids, sent, emb)
    return _tc_stage(s, deg, sent, basis, comp, root, bias, WQ, WK, WV,
                     attW, attb, fcW, fcb)


# speedup vs baseline: 4.0432x; 4.0432x over previous
import jax
import jax.numpy as jnp
from jax import lax
from jax.experimental import pallas as pl
from jax.experimental.pallas import tpu as pltpu
from jax.experimental.pallas import tpu_sc as plsc

B = 4
N_NODES = 10000
E = 160000
L = 50
D = 128
H = 128
HEADS = 3
R = 5
NB = 4
VOCAB = 100000

NC = 2
NS = 16
NW = NC * NS
EPT = E // NW
FULLG = EPT // 16
TAIL = EPT - FULLG * 16
WIN = 5120

SROWS = 304
XBASE = 1 + R * L
CAPL = 32
LP = 64


def _sc_body(ei_hbm, et_hbm, nids_hbm, sent_hbm, emb_hbm, s_out, deg_out,
             marker, nids0, nids1, srcb0, srcb1, dstb0, dstb1, etb0, etb1,
             sentb0, sentb1, idx_s, idx_o, idx_m, tix_s, tix_o, tix_m,
             rowbuf, ones_r, zrows, z16, sg, degg, sem_st, sem_a, sem_b):
    nids = (nids0, nids1)
    srcb = (srcb0, srcb1)
    dstb = (dstb0, dstb1)
    etb = (etb0, etb1)
    sentb = (sentb0, sentb1)
    cc = lax.axis_index("c")
    ss = lax.axis_index("s")
    wid = cc * NS + ss

    zero16f = jnp.zeros((16,), jnp.float32)
    one16f = jnp.ones((16,), jnp.float32)
    zero16i = jnp.zeros((16,), jnp.int32)
    neg16i = jnp.full((16,), -1, jnp.int32)
    lane = lax.iota(jnp.int32, 16)

    abase = pl.multiple_of((wid * EPT >> 7) << 7, 128)
    eoff = wid * EPT - abase

    def stage(g, slot):
        descs = [
            pltpu.async_copy(ei_hbm.at[g, 0, pl.ds(abase, WIN)],
                             srcb[slot].at[pl.ds(0, WIN)], sem_st),
            pltpu.async_copy(ei_hbm.at[g, 1, pl.ds(abase, WIN)],
                             dstb[slot].at[pl.ds(0, WIN)], sem_st),
            pltpu.async_copy(et_hbm.at[g, pl.ds(abase, WIN)],
                             etb[slot].at[pl.ds(0, WIN)], sem_st),
            pltpu.async_copy(nids_hbm.at[g], nids[slot], sem_st),
            pltpu.async_copy(sent_hbm.at[g], sentb[slot], sem_st),
        ]
        return descs

    def stage_wait(descs):
        for d in descs:
            d.wait()

    descs = stage(0, 0)

    def _init_const(i, _):
        for k in range(8):
            zrows[i, pl.ds(k * 16, 16)] = zero16f
        z16[i, pl.ds(0, 16)] = zero16f
        ones_r[i, pl.ds(0, 16)] = one16f
        ones_r[i + 16, pl.ds(0, 16)] = one16f
        return 0
    lax.fori_loop(0, 16, _init_const, 0)

    def _memset(i, _):
        marker[pl.ds(i * 16, 16)] = neg16i
        return 0
    lax.fori_loop(0, N_NODES // 16, _memset, 0)

    idx = 0
    for g in range(B):
        for ch in range(SROWS // 16):
            @pl.when(ss == (idx % NS))
            def _():
                pltpu.sync_copy(zrows, sg.at[g, pl.ds(ch * 16, 16)])
            idx += 1
    for g in range(B):
        for ch in range(LP // 16):
            @pl.when(ss == ((g * (LP // 16) + ch) % NS))
            def _():
                pltpu.sync_copy(z16, degg.at[g, pl.ds(ch * 16, 16)])

    stage_wait(descs)
    plsc.subcore_barrier()

    for g in range(B):
        slot = g % 2
        tv = dstb[slot][pl.ds(eoff + FULLG * 16, 16)]
        dstb[slot][pl.ds(eoff + FULLG * 16, 16)] = jnp.where(lane < TAIL, tv, 0)
        sv = srcb[slot][pl.ds(eoff + FULLG * 16, 16)]
        srcb[slot][pl.ds(eoff + FULLG * 16, 16)] = jnp.where(lane < TAIL, sv, 0)

        for k in range(LP // 16):
            se = sentb[slot][pl.ds(k * 16, 16)]
            lv = lane + k * 16
            plsc.store_scatter(marker, [se], lv, mask=lv < L)

        def _prefill(ch, _):
            for k in range(2):
                idx_s[ch, pl.ds(k * 16, 16)] = zero16i
                idx_o[ch, pl.ds(k * 16, 16)] = zero16i
                idx_m[ch, pl.ds(k * 16, 16)] = zero16i
            return 0
        lax.fori_loop(0, CAPL, _prefill, 0)

        def _scan_group(j, cnt, col, tail_mask=None):
            d = dstb[slot][pl.ds(eoff + j * 16, 16)]
            s = srcb[slot][pl.ds(eoff + j * 16, 16)]
            t = etb[slot][pl.ds(eoff + j * 16, 16)]
            m = plsc.load_gather(marker, [d])
            valid = m >= 0
            if tail_mask is not None:
                valid = valid & tail_mask
            nid = plsc.load_gather(nids[slot], [s])
            oidx = 1 + t * L + m
            midx = 1 + m
            pos = jnp.minimum(cnt, CAPL - 1)
            plsc.store_scatter(idx_s, [pos, col], nid, mask=valid)
            plsc.store_scatter(idx_o, [pos, col], oidx, mask=valid)
            plsc.store_scatter(idx_m, [pos, col], midx, mask=valid)
            return jnp.minimum(cnt + valid.astype(jnp.int32), CAPL)

        lane16 = lane + 16

        def _scan_pair(p, carry):
            ca, cb = carry
            ca = _scan_group(2 * p, ca, lane)
            cb = _scan_group(2 * p + 1, cb, lane16)
            return (ca, cb)

        cnt_a, cnt_b = lax.fori_loop(
            0, FULLG // 2, _scan_pair,
            (jnp.zeros((16,), jnp.int32), jnp.zeros((16,), jnp.int32)))
        cnt_a = _scan_group(FULLG, cnt_a, lane, tail_mask=lane < TAIL)

        for k in range(LP // 16):
            se = sentb[slot][pl.ds(k * 16, 16)]
            lv = lane + k * 16
            plsc.store_scatter(marker, [se], neg16i, mask=lv < L)

        t0 = wid == 0
        for k in range(LP // 16):
            se = sentb[slot][pl.ds(k * 16, 16)]
            lv = lane + k * 16
            valid = (lv < L) & t0
            nid = plsc.load_gather(nids[slot], [se])
            pos = jnp.minimum(cnt_a, CAPL - 1)
            plsc.store_scatter(idx_s, [pos, lane], nid, mask=valid)
            plsc.store_scatter(idx_o, [pos, lane], XBASE + lv, mask=valid)
            plsc.store_scatter(idx_m, [pos, lane], zero16i, mask=valid)
            cnt_a = jnp.minimum(cnt_a + valid.astype(jnp.int32), CAPL)

        if g + 1 < B:
            descs = stage(g + 1, (g + 1) % 2)

        ca_cum = plsc.cumsum(cnt_a)
        tot_a = jnp.max(ca_cum)
        cb_cum = plsc.cumsum(cnt_b)
        tot = tot_a + jnp.max(cb_cum)
        start_a = ca_cum - cnt_a
        start_b = tot_a + cb_cum - cnt_b
        mv_n = jnp.maximum(jnp.max(cnt_a), jnp.max(cnt_b))

        def _prefill_t(ch, _):
            for k in range(2):
                tix_s[ch, pl.ds(k * 16, 16)] = zero16i
                tix_o[ch, pl.ds(k * 16, 16)] = zero16i
                tix_m[ch, pl.ds(k * 16, 16)] = zero16i
            return 0
        lax.fori_loop(0, CAPL, _prefill_t, 0)

        def _move(r, _):
            for k, (st, cn) in enumerate(((start_a, cnt_a), (start_b, cnt_b))):
                dest = st + r
                hi = dest >> 5
                lo = dest & 31
                msk = r < cn
                vs = idx_s[r, pl.ds(k * 16, 16)]
                vo = idx_o[r, pl.ds(k * 16, 16)]
                vm = idx_m[r, pl.ds(k * 16, 16)]
                plsc.store_scatter(tix_s, [hi, lo], vs, mask=msk)
                plsc.store_scatter(tix_o, [hi, lo], vo, mask=msk)
                plsc.store_scatter(tix_m, [hi, lo], vm, mask=msk)
            return 0
        lax.fori_loop(0, mv_n, _move, 0)

        nch = (tot + 31) >> 5

        def _flush(j, _):
            pltpu.async_copy(emb_hbm.at[tix_s.at[j]], rowbuf, sem_a).wait()
            pltpu.sync_copy(rowbuf, sg.at[g].at[tix_o.at[j]], add=True)
            pltpu.sync_copy(ones_r, degg.at[g].at[tix_m.at[j]], add=True)
            return 0
        lax.fori_loop(0, nch, _flush, 0)

        if g + 1 < B:
            stage_wait(descs)

    plsc.subcore_barrier()

    idx = 0
    for g in range(B):
        for ch in range(SROWS // 16):
            @pl.when(ss == (idx % NS))
            def _():
                pltpu.sync_copy(sg.at[g, pl.ds(ch * 16, 16)],
                                s_out.at[cc, g, pl.ds(ch * 16, 16)])
            idx += 1
    for g in range(B):
        for ch in range(LP // 16):
            @pl.when(ss == ((g * (LP // 16) + ch) % NS))
            def _():
                pltpu.sync_copy(degg.at[g, pl.ds(ch * 16, 16)],
                                deg_out.at[cc, g, pl.ds(ch * 16, 16)])


@jax.jit
def _sc_stage(ei, et, nids, sent, emb):
    mesh = plsc.VectorSubcoreMesh(core_axis_name="c", subcore_axis_name="s")
    f = pl.kernel(
        _sc_body,
        out_type=[
            jax.ShapeDtypeStruct((NC, B, SROWS, D), jnp.float32),
            jax.ShapeDtypeStruct((NC, B, LP, 16), jnp.float32),
        ],
        mesh=mesh,
        compiler_params=pltpu.CompilerParams(needs_layout_passes=False),
        scratch_types=[
            pltpu.VMEM((N_NODES,), jnp.int32),
            pltpu.VMEM((N_NODES,), jnp.int32),
            pltpu.VMEM((N_NODES,), jnp.int32),
            pltpu.VMEM((WIN + 16,), jnp.int32),
            pltpu.VMEM((WIN + 16,), jnp.int32),
            pltpu.VMEM((WIN + 16,), jnp.int32),
            pltpu.VMEM((WIN + 16,), jnp.int32),
            pltpu.VMEM((WIN + 16,), jnp.int32),
            pltpu.VMEM((WIN + 16,), jnp.int32),
            pltpu.VMEM((LP,), jnp.int32),
            pltpu.VMEM((LP,), jnp.int32),
            pltpu.VMEM((CAPL, 32), jnp.int32),
            pltpu.VMEM((CAPL, 32), jnp.int32),
            pltpu.VMEM((CAPL, 32), jnp.int32),
            pltpu.VMEM((CAPL, 32), jnp.int32),
            pltpu.VMEM((CAPL, 32), jnp.int32),
            pltpu.VMEM((CAPL, 32), jnp.int32),
            pltpu.VMEM((32, D), jnp.float32),
            pltpu.VMEM((32, 16), jnp.float32),
            pltpu.VMEM((16, D), jnp.float32),
            pltpu.VMEM((16, 16), jnp.float32),
            pltpu.VMEM_SHARED((B, SROWS, D), jnp.float32),
            pltpu.VMEM_SHARED((B, LP, 16), jnp.float32),
            pltpu.SemaphoreType.DMA,
            pltpu.SemaphoreType.DMA,
            pltpu.SemaphoreType.DMA,
        ],
    )
    return f(ei, et, nids, sent, emb)


def _tc_body(s_ref, deg_ref, sent_ref, basis_ref, comp_ref, root_ref, bias_ref,
             wq_ref, wk_ref, wv_ref, attw_ref, attb_ref, fcw_ref, fcb_ref,
             out_ref):
    s = s_ref[0] + s_ref[1]
    deg = jnp.sum(deg_ref[0] + deg_ref[1], axis=-1) * (1.0 / 16.0)

    w = jnp.einsum("rb,bde->rde", comp_ref[...], basis_ref[...])
    msum = jnp.zeros((B, L, D), jnp.float32)
    for r in range(R):
        msum = msum + jnp.einsum("gld,de->gle", s[:, 1 + r * L:1 + (r + 1) * L, :], w[r])

    sent = sent_ref[...][:, :L]
    p = (sent[:, :, None] == sent[:, None, :]).astype(jnp.float32)
    sum_s = jnp.einsum("glk,gkd->gld", p, msum)
    sum_deg = jnp.einsum("glk,gk->gl", p, deg[:, 1:1 + L])
    agg = sum_s / jnp.clip(sum_deg, 1.0, None)[..., None]

    xs = s[:, XBASE:XBASE + L, :]
    x = jnp.einsum("gld,de->gle", xs, root_ref[...]) + agg + bias_ref[...]

    def heads(wref):
        hh = jnp.einsum("gld,hde->glhe", x, wref[...])
        return hh.reshape(B, L, HEADS * H)
    q = heads(wq_ref)
    k = heads(wk_ref)
    v = heads(wv_ref)
    score = jnp.einsum("gle,gme->glm", q, k) * (1.0 / jnp.sqrt(float(HEADS * H)))
    score = jax.nn.softmax(score, axis=-1)
    hidden = jnp.einsum("glm,gme->gle", score, v)
    attn = jnp.einsum("gle,eo->glo", hidden, attw_ref[...]) + attb_ref[...]
    pooled = jnp.sum(hidden * attn, axis=1)
    logits = pooled @ fcw_ref[...] + fcb_ref[...]
    out_ref[...] = 1.0 / (1.0 + jnp.exp(-logits))


@jax.jit
def _tc_stage(s, deg, sent, basis, comp, root, bias, wq, wk, wv, attw, attb,
              fcw, fcb):
    return pl.pallas_call(
        _tc_body,
        out_shape=jax.ShapeDtypeStruct((B, 1), jnp.float32),
    )(s, deg, sent, basis, comp, root, bias, wq, wk, wv, attw, attb, fcw, fcb)


def kernel(sent_ids, edge_index, edge_type, node_ids, emb, basis, comp, root,
           bias, WQ, WK, WV, attW, attb, fcW, fcb):
    ei = edge_index.astype(jnp.int32)
    et = edge_type.astype(jnp.int32)
    sent = jnp.pad(sent_ids.astype(jnp.int32), ((0, 0), (0, LP - L)))
    nids = node_ids.astype(jnp.int32)
    s, deg = _sc_stage(ei, et, nids, sent, emb)
    return _tc_stage(s, deg, sent, basis, comp, root, bias, WQ, WK, WV,
                     attW, attb, fcW, fcb)
